# Initial kernel scaffold; baseline (speedup 1.0000x reference)
import functools

import jax
import jax.numpy as jnp
from jax import lax
from jax.experimental import pallas as pl
from jax.experimental.pallas import tpu as pltpu
from jax.experimental.pallas import tpu_sc as plsc

N = 8388608
TABLE_WORDS = 1 << 19
NC, NS, L = 2, 16, 16
NW = NC * NS
PER_W = N // NW
CHUNK = 2048
SUB = 128
NSUB = CHUNK // SUB
NCHUNK = PER_W // CHUNK
VEC_ITERS = CHUNK // L

_mesh = plsc.VectorSubcoreMesh(core_axis_name="c", subcore_axis_name="s")


@functools.partial(
    pl.kernel,
    out_type=jax.ShapeDtypeStruct((N,), jnp.int32),
    mesh=_mesh,
    scratch_types=[
        pltpu.VMEM_SHARED((TABLE_WORDS,), jnp.int32),
        pltpu.VMEM((CHUNK, 2), jnp.int32),
        pltpu.VMEM((CHUNK,), jnp.int32),
        pltpu.VMEM((CHUNK,), jnp.int32),
        pltpu.VMEM((CHUNK,), jnp.int32),
        pltpu.VMEM((CHUNK,), jnp.int32),
        pltpu.SemaphoreType.DMA,
    ],
)
def _lcg_probe(x2_hbm, table_hbm, out_hbm, table_sp, in_buf, widx_buf,
               bpos_buf, words_buf, out_buf, sem):
    cid = lax.axis_index("c")
    sid = lax.axis_index("s")
    wid = sid * NC + cid

    @pl.when(sid == 0)
    def _():
        pltpu.sync_copy(table_hbm, table_sp)

    plsc.subcore_barrier()

    iota = lax.broadcasted_iota(jnp.int32, (L,), 0)
    ones = jnp.ones((L,), jnp.int32)
    base_w = wid * PER_W

    def chunk_body(g, carry):
        base = base_w + g * CHUNK
        pltpu.sync_copy(x2_hbm.at[pl.ds(base, CHUNK)], in_buf)

        def vec1(i, c):
            rows = iota + i * L
            h = plsc.load_gather(in_buf, [rows, ones])
            widx_buf[pl.ds(i * L, L)] = lax.shift_right_logical(h, 12)
            bpos_buf[pl.ds(i * L, L)] = lax.shift_right_logical(h, 7) & 31
            return c

        lax.fori_loop(0, VEC_ITERS, vec1, 0)

        copies = []
        for j in range(NSUB):
            idx = widx_buf.at[pl.ds(j * SUB, SUB)]
            copies.append(pltpu.async_copy(
                table_sp.at[idx], words_buf.at[pl.ds(j * SUB, SUB)], sem))
        for c in copies:
            c.wait()

        def vec2(i, c):
            sl = pl.ds(i * L, L)
            w = words_buf[sl]
            bp = bpos_buf[sl]
            out_buf[sl] = lax.shift_right_logical(w, bp) & 1
            return c

        lax.fori_loop(0, VEC_ITERS, vec2, 0)
        pltpu.sync_copy(out_buf, out_hbm.at[pl.ds(base, CHUNK)])
        return carry

    lax.fori_loop(0, NCHUNK, chunk_body, 0)


def kernel(binary_set, x, is_training, test_local_stats):
    x2 = lax.bitcast_convert_type(x, jnp.int32)
    table = lax.bitcast_convert_type(
        binary_set.reshape(TABLE_WORDS, 4), jnp.int32)
    out = _lcg_probe(x2, table)
    return out.astype(jnp.bool_)


# speedup vs baseline: 7.6684x; 7.6684x over previous
import functools

import jax
import jax.numpy as jnp
from jax import lax
from jax.experimental import pallas as pl
from jax.experimental.pallas import tpu as pltpu
from jax.experimental.pallas import tpu_sc as plsc

N = 8388608
TABLE_WORDS = 1 << 19
NC, NS, L = 2, 16, 16
NW = NC * NS
PER_W = N // NW
CHUNK = 2048
SUB = 128
NSUB = CHUNK // SUB
NCHUNK = PER_W // CHUNK
VEC_ITERS = CHUNK // L

_mesh = plsc.VectorSubcoreMesh(core_axis_name="c", subcore_axis_name="s")


def _c(v):
    return jnp.int32(v)


@functools.partial(
    pl.kernel,
    out_type=jax.ShapeDtypeStruct((N,), jnp.int32),
    mesh=_mesh,
    scratch_types=[
        pltpu.VMEM_SHARED((TABLE_WORDS,), jnp.int32),
        pltpu.VMEM((2 * CHUNK,), jnp.int32),
        pltpu.VMEM((CHUNK,), jnp.int32),
        pltpu.VMEM((CHUNK,), jnp.int32),
        pltpu.VMEM((CHUNK,), jnp.int32),
        pltpu.VMEM((CHUNK,), jnp.int32),
        pltpu.SemaphoreType.DMA,
    ],
    compiler_params=pltpu.CompilerParams(needs_layout_passes=False),
)
def _lcg_probe(x2_hbm, table_hbm, out_hbm, table_sp, in_buf, widx_buf,
               bpos_buf, words_buf, out_buf, sem):
    cid = lax.axis_index("c")
    sid = lax.axis_index("s")
    wid = sid * _c(NC) + cid

    @pl.when(sid == 0)
    def _():
        pltpu.sync_copy(table_hbm, table_sp)

    plsc.subcore_barrier()

    iota = lax.broadcasted_iota(jnp.int32, (L,), 0)
    base_w = wid * _c(PER_W)

    def chunk_body(g, carry):
        base = base_w + g * _c(CHUNK)
        pltpu.sync_copy(x2_hbm.at[pl.ds(base * _c(2), 2 * CHUNK)], in_buf)

        def vec1(i, c):
            rows = (iota + i * _c(L)) * _c(2) + _c(1)
            h = plsc.load_gather(in_buf, [rows])
            widx_buf[pl.ds(i * _c(L), L)] = lax.shift_right_logical(h, _c(12))
            bpos_buf[pl.ds(i * _c(L), L)] = (
                lax.shift_right_logical(h, _c(7)) & _c(31))
            return c

        lax.fori_loop(_c(0), _c(VEC_ITERS), vec1, _c(0))

        copies = []
        for j in range(NSUB):
            idx = widx_buf.at[pl.ds(j * SUB, SUB)]
            copies.append(pltpu.async_copy(
                table_sp.at[idx], words_buf.at[pl.ds(j * SUB, SUB)], sem))
        for c in copies:
            c.wait()

        def vec2(i, c):
            sl = pl.ds(i * _c(L), L)
            w = words_buf[sl]
            bp = bpos_buf[sl]
            out_buf[sl] = lax.shift_right_logical(w, bp) & _c(1)
            return c

        lax.fori_loop(_c(0), _c(VEC_ITERS), vec2, _c(0))
        pltpu.sync_copy(out_buf, out_hbm.at[pl.ds(base, CHUNK)])
        return carry

    lax.fori_loop(_c(0), _c(NCHUNK), chunk_body, _c(0))


def kernel(binary_set, x, is_training, test_local_stats):
    x2 = lax.bitcast_convert_type(x, jnp.int32).reshape(2 * N)
    table = lax.bitcast_convert_type(
        binary_set.reshape(TABLE_WORDS, 4), jnp.int32)
    out = _lcg_probe(x2, table)
    return out.astype(jnp.bool_)
